# Initial kernel scaffold; baseline (speedup 1.0000x reference)
import jax
import jax.numpy as jnp
from jax.experimental import pallas as pl


def kernel(input_data, num_active_vectors, previous_active_vectors, codebook):
    raise NotImplementedError("write your pallas kernel here")


# speedup vs baseline: 2.5644x; 2.5644x over previous
import functools

import jax
import jax.numpy as jnp
from jax import lax
from jax.experimental import pallas as pl
from jax.experimental.pallas import tpu as pltpu

_D = 64
_A = 16
_B = 16
_HW = 1024
_NTOK = _B * _HW
_CLW = 0.1
_PLW = 0.33


def _avq_kernel(x_ref, cb_ref, cbt_ref, prev_ref, quant_ref, loss_ref):
    b = pl.program_id(0)
    nb = pl.num_programs(0)
    x = x_ref[0]
    cb = cb_ref[...]
    cbt = cbt_ref[...]

    dot = jax.lax.dot_general(cb, x, (((1,), (0,)), ((), ())),
                              preferred_element_type=jnp.float32)
    xnorm = jnp.sum(x * x, axis=0, keepdims=True)
    cbnorm = jnp.sum(cb * cb, axis=1, keepdims=True)
    dist = (xnorm + cbnorm) - 2.0 * dot

    rowid = lax.broadcasted_iota(jnp.int32, (_A, _HW), 0)
    part = jnp.zeros((1, 128), jnp.float32)
    lane = lax.broadcasted_iota(jnp.int32, (1, 128), 1)
    for lvl in range(4):
        a = 2 ** (lvl + 1)
        big = jnp.where(rowid < a, dist, jnp.inf)
        minv = jnp.min(big, axis=0, keepdims=True)
        idx = jnp.min(jnp.where(big == minv, rowid, _A), axis=0,
                      keepdims=True)
        onehot = (rowid == idx).astype(jnp.float32)
        q = jax.lax.dot_general(cbt, onehot, (((1,), (0,)), ((), ())),
                                preferred_element_type=jnp.float32)
        quant_ref[lvl, 0] = x + (q - x)
        d = q - x
        part = part + jnp.where(lane == lvl, jnp.sum(d * d), 0.0)

    @pl.when(b == 0)
    def _init():
        loss_ref[...] = jnp.zeros((1, 128), jnp.float32)

    acc = loss_ref[...] + part

    m = acc * jnp.float32(1.0 / (_NTOK * _D))
    prev = prev_ref[...]
    dv = prev - cb
    dv2 = dv * dv
    prow = lax.broadcasted_iota(jnp.int32, (_A, _D), 0)
    p2 = jnp.sum(jnp.where(prow < 2, dv2, 0.0)) * jnp.float32(1.0 / (2 * _D))
    p4 = jnp.sum(jnp.where(prow < 4, dv2, 0.0)) * jnp.float32(1.0 / (4 * _D))
    p8 = jnp.sum(jnp.where(prow < 8, dv2, 0.0)) * jnp.float32(1.0 / (8 * _D))
    clw_vec = jnp.where(lane < 2, jnp.float32(_CLW), 0.0)
    prox_vec = (jnp.where(lane == 1, jnp.float32(1 * _PLW) * p2, 0.0)
                + jnp.where(lane == 2, jnp.float32(_PLW) * p4, 0.0)
                + jnp.where(lane == 3, jnp.float32(_PLW) * p8, 0.0))
    final = (m + clw_vec * m) + prox_vec

    loss_ref[...] = jnp.where(b == nb - 1, final, acc)


@jax.jit
def _run(x3, cb16, cbt16, prev):
    quant, loss = pl.pallas_call(
        _avq_kernel,
        grid=(_B,),
        in_specs=[
            pl.BlockSpec((1, _D, _HW), lambda b: (b, 0, 0)),
            pl.BlockSpec((_A, _D), lambda b: (0, 0)),
            pl.BlockSpec((_D, _A), lambda b: (0, 0)),
            pl.BlockSpec((_A, _D), lambda b: (0, 0)),
        ],
        out_specs=[
            pl.BlockSpec((4, 1, _D, _HW), lambda b: (0, b, 0, 0)),
            pl.BlockSpec((1, 128), lambda b: (0, 0)),
        ],
        out_shape=[
            jax.ShapeDtypeStruct((4, _B, _D, _HW), jnp.float32),
            jax.ShapeDtypeStruct((1, 128), jnp.float32),
        ],
        compiler_params=pltpu.CompilerParams(
            dimension_semantics=("arbitrary",)),
    )(x3, cb16, cbt16, prev)
    return quant, loss


def kernel(input_data, num_active_vectors, previous_active_vectors, codebook):
    x3 = input_data.reshape(_B, _D, _HW)
    cb16 = codebook[:_A]
    quant, loss = _run(x3, cb16, cb16.T, previous_active_vectors)
    quantized = quant.reshape(4, _B, _D, 32, 32)
    losses = loss[0, :4]
    return quantized, losses, cb16


# speedup vs baseline: 2.9093x; 1.1345x over previous
import functools

import jax
import jax.numpy as jnp
from jax import lax
from jax.experimental import pallas as pl
from jax.experimental.pallas import tpu as pltpu

_D = 64
_A = 16
_B = 16
_HW = 1024
_NTOK = _B * _HW
_CLW = 0.1
_PLW = 0.33


def _avq_kernel(x_ref, cb_ref, cbt_ref, prev_ref, quant_ref, loss_ref):
    b = pl.program_id(0)
    nb = pl.num_programs(0)
    cb = cb_ref[...]
    cbt = cbt_ref[...]
    cbnorm = jnp.sum(cb * cb, axis=1, keepdims=True)

    part = jnp.zeros((1, 128), jnp.float32)
    lane = lax.broadcasted_iota(jnp.int32, (1, 128), 1)
    rowid = lax.broadcasted_iota(jnp.int32, (_A, _HW), 0)
    for sb in range(_BB):
        x = x_ref[sb]
        dot = jax.lax.dot_general(cb, x, (((1,), (0,)), ((), ())),
                                  preferred_element_type=jnp.float32)
        xnorm = jnp.sum(x * x, axis=0, keepdims=True)
        dist = (xnorm + cbnorm) - 2.0 * dot

        for lvl in range(4):
            a = 2 ** (lvl + 1)
            big = jnp.where(rowid < a, dist, jnp.inf)
            minv = jnp.min(big, axis=0, keepdims=True)
            idx = jnp.min(jnp.where(big == minv, rowid, _A), axis=0,
                          keepdims=True)
            onehot = (rowid == idx).astype(jnp.float32)
            q = jax.lax.dot_general(cbt, onehot, (((1,), (0,)), ((), ())),
                                    preferred_element_type=jnp.float32)
            quant_ref[lvl, sb] = x + (q - x)
            d = q - x
            part = part + jnp.where(lane == lvl, jnp.sum(d * d), 0.0)

    @pl.when(b == 0)
    def _init():
        loss_ref[...] = jnp.zeros((1, 128), jnp.float32)

    acc = loss_ref[...] + part

    m = acc * jnp.float32(1.0 / (_NTOK * _D))
    prev = prev_ref[...]
    dv = prev - cb
    dv2 = dv * dv
    prow = lax.broadcasted_iota(jnp.int32, (_A, _D), 0)
    p2 = jnp.sum(jnp.where(prow < 2, dv2, 0.0)) * jnp.float32(1.0 / (2 * _D))
    p4 = jnp.sum(jnp.where(prow < 4, dv2, 0.0)) * jnp.float32(1.0 / (4 * _D))
    p8 = jnp.sum(jnp.where(prow < 8, dv2, 0.0)) * jnp.float32(1.0 / (8 * _D))
    clw_vec = jnp.where(lane < 2, jnp.float32(_CLW), 0.0)
    prox_vec = (jnp.where(lane == 1, jnp.float32(1 * _PLW) * p2, 0.0)
                + jnp.where(lane == 2, jnp.float32(_PLW) * p4, 0.0)
                + jnp.where(lane == 3, jnp.float32(_PLW) * p8, 0.0))
    final = (m + clw_vec * m) + prox_vec

    loss_ref[...] = jnp.where(b == nb - 1, final, acc)


_BB = 4


@jax.jit
def _run(x3, cb16, cbt16, prev):
    quant, loss = pl.pallas_call(
        _avq_kernel,
        grid=(_B // _BB,),
        in_specs=[
            pl.BlockSpec((_BB, _D, _HW), lambda b: (b, 0, 0)),
            pl.BlockSpec((_A, _D), lambda b: (0, 0)),
            pl.BlockSpec((_D, _A), lambda b: (0, 0)),
            pl.BlockSpec((_A, _D), lambda b: (0, 0)),
        ],
        out_specs=[
            pl.BlockSpec((4, _BB, _D, _HW), lambda b: (0, b, 0, 0)),
            pl.BlockSpec((1, 128), lambda b: (0, 0)),
        ],
        out_shape=[
            jax.ShapeDtypeStruct((4, _B, _D, _HW), jnp.float32),
            jax.ShapeDtypeStruct((1, 128), jnp.float32),
        ],
        compiler_params=pltpu.CompilerParams(
            dimension_semantics=("arbitrary",)),
    )(x3, cb16, cbt16, prev)
    return quant, loss


def kernel(input_data, num_active_vectors, previous_active_vectors, codebook):
    x3 = input_data.reshape(_B, _D, _HW)
    cb16 = codebook[:_A]
    quant, loss = _run(x3, cb16, cb16.T, previous_active_vectors)
    quantized = quant.reshape(4, _B, _D, 32, 32)
    losses = loss[0, :4]
    return quantized, losses, cb16
